# baseline (speedup 1.0000x reference)
import functools

import jax
import jax.numpy as jnp
from jax import lax
from jax.experimental import pallas as pl
from jax.experimental.pallas import tpu as pltpu
from jax.experimental.pallas import tpu_sc as plsc

N = 10000
D = 128
E = 320000

NC = 2
NS = 16
NW = NC * NS
L = 16

B = 128
G = 80
EPW = G * B
EP = NW * EPW
GH = G // 2
ROWS = 10112
RPT = ROWS // NS

_sc_mesh = plsc.VectorSubcoreMesh(core_axis_name="c", subcore_axis_name="s")


@functools.partial(
    pl.kernel,
    out_type=jax.ShapeDtypeStruct((NC * 2 * ROWS,), jnp.float32),
    mesh=_sc_mesh,
    scratch_types=[
        pltpu.VMEM((G, B), jnp.int32),
        pltpu.VMEM((G, B), jnp.int32),
        pltpu.VMEM((B,), jnp.float32),
        pltpu.VMEM((640,), jnp.float32),
        pltpu.SemaphoreType.DMA,
        pltpu.SemaphoreType.DMA,
        pltpu.VMEM_SHARED((ROWS,), jnp.float32),
        pltpu.VMEM_SHARED((ROWS,), jnp.float32),
    ],
)
def _deg_kernel(src_hbm, dst_hbm, out_hbm, sidx, didx, ones, zbuf,
                sema, semb, acc_o, acc_i):
    c = lax.axis_index("c")
    s = lax.axis_index("s")
    wid = s * NC + c

    vone = jnp.ones((L,), jnp.float32)
    vz = jnp.zeros((L,), jnp.float32)
    for j in range(B // L):
        ones[pl.ds(j * L, L)] = vone
    for j in range(640 // L):
        zbuf[pl.ds(j * L, L)] = vz

    r0 = s * RPT
    pltpu.sync_copy(zbuf.at[pl.ds(0, RPT)], acc_o.at[pl.ds(r0, RPT)])
    pltpu.sync_copy(zbuf.at[pl.ds(0, RPT)], acc_i.at[pl.ds(r0, RPT)])
    pltpu.sync_copy(src_hbm.at[pl.ds(wid * G, G)], sidx)
    pltpu.sync_copy(dst_hbm.at[pl.ds(wid * G, G)], didx)
    plsc.subcore_barrier()

    pltpu.async_copy(ones, acc_o.at[sidx.at[0]], sema, add=True)
    pltpu.async_copy(ones, acc_i.at[didx.at[0]], semb, add=True)

    def body(g, _):
        pltpu.async_copy(ones, acc_o.at[sidx.at[g]], sema, add=True)
        pltpu.async_copy(ones, acc_i.at[didx.at[g]], semb, add=True)
        pltpu.make_async_copy(ones, acc_o.at[sidx.at[g]], sema).wait()
        pltpu.make_async_copy(ones, acc_i.at[didx.at[g]], semb).wait()
        return 0

    lax.fori_loop(1, G, body, 0)
    pltpu.make_async_copy(ones, acc_o.at[sidx.at[0]], sema).wait()
    pltpu.make_async_copy(ones, acc_i.at[didx.at[0]], semb).wait()
    plsc.subcore_barrier()

    pltpu.sync_copy(acc_o.at[pl.ds(r0, RPT)], zbuf.at[pl.ds(0, RPT)])
    pltpu.sync_copy(zbuf.at[pl.ds(0, RPT)],
                    out_hbm.at[pl.ds((c * 2) * ROWS + r0, RPT)])
    pltpu.sync_copy(acc_i.at[pl.ds(r0, RPT)], zbuf.at[pl.ds(0, RPT)])
    pltpu.sync_copy(zbuf.at[pl.ds(0, RPT)],
                    out_hbm.at[pl.ds((c * 2 + 1) * ROWS + r0, RPT)])


@functools.partial(
    pl.kernel,
    out_type=jax.ShapeDtypeStruct((NC, ROWS, D), jnp.float32),
    mesh=_sc_mesh,
    scratch_types=[
        pltpu.VMEM((GH, B), jnp.int32),
        pltpu.VMEM((GH, B), jnp.int32),
        pltpu.VMEM((B, D), jnp.float32),
        pltpu.VMEM((B, D), jnp.float32),
        pltpu.SemaphoreType.DMA,
        pltpu.SemaphoreType.DMA,
        pltpu.SemaphoreType.DMA,
        pltpu.SemaphoreType.DMA,
        pltpu.VMEM_SHARED((ROWS, D), jnp.float32),
    ],
)
def _agg_kernel(h_hbm, src_hbm, dst_hbm, z_hbm, out_hbm,
                sidx, didx, rows0, rows1, sem0, sem1, ssem0, ssem1, acc):
    c = lax.axis_index("c")
    s = lax.axis_index("s")
    wid = s * NC + c

    r0 = s * RPT
    pltpu.sync_copy(z_hbm, rows0)
    for p in range(RPT // B):
        pltpu.sync_copy(rows0, acc.at[pl.ds(r0 + p * B, B)])
    rem = RPT % B
    if rem:
        pltpu.sync_copy(rows0.at[pl.ds(0, rem)],
                        acc.at[pl.ds(r0 + (RPT // B) * B, rem)])
    plsc.subcore_barrier()

    for half in range(2):
        pltpu.sync_copy(src_hbm.at[pl.ds(wid * G + half * GH, GH)], sidx)
        pltpu.sync_copy(dst_hbm.at[pl.ds(wid * G + half * GH, GH)], didx)

        pltpu.async_copy(h_hbm.at[sidx.at[0]], rows0, sem0)

        def body(k, _):
            g = 2 * k
            pltpu.async_copy(h_hbm.at[sidx.at[g + 1]], rows1, sem1)
            pltpu.make_async_copy(h_hbm.at[sidx.at[g]], rows0, sem0).wait()
            pltpu.sync_copy(rows0, acc.at[didx.at[g]], add=True)
            pltpu.async_copy(h_hbm.at[sidx.at[g + 2]], rows0, sem0)
            pltpu.make_async_copy(h_hbm.at[sidx.at[g + 1]], rows1, sem1).wait()
            pltpu.sync_copy(rows1, acc.at[didx.at[g + 1]], add=True)
            return 0

        lax.fori_loop(0, GH // 2 - 1, body, 0)
        pltpu.async_copy(h_hbm.at[sidx.at[GH - 1]], rows1, sem1)
        pltpu.make_async_copy(h_hbm.at[sidx.at[GH - 2]], rows0, sem0).wait()
        pltpu.sync_copy(rows0, acc.at[didx.at[GH - 2]], add=True)
        pltpu.make_async_copy(h_hbm.at[sidx.at[GH - 1]], rows1, sem1).wait()
        pltpu.sync_copy(rows1, acc.at[didx.at[GH - 1]], add=True)

    plsc.subcore_barrier()
    for p in range(RPT // B):
        pltpu.sync_copy(acc.at[pl.ds(r0 + p * B, B)], rows0)
        pltpu.sync_copy(rows0, out_hbm.at[c, pl.ds(r0 + p * B, B)])
    if RPT % B:
        q = (RPT // B) * B
        pltpu.sync_copy(acc.at[pl.ds(r0 + q, RPT % B)],
                        rows0.at[pl.ds(0, RPT % B)])
        pltpu.sync_copy(rows0.at[pl.ds(0, RPT % B)],
                        out_hbm.at[c, pl.ds(r0 + q, RPT % B)])


BR = 1264
GRID = ROWS // BR


def _prep_body(degp_ref, x_ref, h1s_ref, nin_ref, nout_ref):
    dt = degp_ref[...]
    deg_out = dt[:, 0:1] + dt[:, 2:3] + 1.0
    deg_in = dt[:, 1:2] + dt[:, 3:4] + 1.0
    nout = lax.rsqrt(deg_out)
    nin = lax.rsqrt(deg_in)
    h1s_ref[...] = x_ref[...] * nout
    nin_ref[...] = jnp.broadcast_to(nin, (ROWS, D))
    nout_ref[...] = jnp.broadcast_to(nout, (ROWS, D))


def _tc_prep(degp2, xp):
    return pl.pallas_call(
        _prep_body,
        out_shape=[
            jax.ShapeDtypeStruct((ROWS, D), jnp.float32),
            jax.ShapeDtypeStruct((ROWS, D), jnp.float32),
            jax.ShapeDtypeStruct((ROWS, D), jnp.float32),
        ],
    )(degp2, xp)


def _combine_body(accp_ref, selfh_ref, nin_ref, nout_ref, w_ref, b_ref,
                  out_ref, *, last, br):
    agg = accp_ref[0] + accp_ref[1] + selfh_ref[...]
    t = agg * nin_ref[...]
    y = jnp.dot(t, w_ref[...], preferred_element_type=jnp.float32) + b_ref[...]
    if last:
        out_ref[...] = y
    else:
        out_ref[...] = jnp.maximum(y, 0.0) * nout_ref[...]


def _tc_combine(accp, selfh, nin, nout, w, b2d, last):
    nrows, br = (N, 1000) if last else (ROWS, BR)
    return pl.pallas_call(
        functools.partial(_combine_body, last=last, br=br),
        grid=(nrows // br,),
        in_specs=[
            pl.BlockSpec((NC, br, D), lambda i: (0, i, 0)),
            pl.BlockSpec((br, D), lambda i: (i, 0)),
            pl.BlockSpec((br, D), lambda i: (i, 0)),
            pl.BlockSpec((br, D), lambda i: (i, 0)),
            pl.BlockSpec((D, D), lambda i: (0, 0)),
            pl.BlockSpec((1, D), lambda i: (0, 0)),
        ],
        out_specs=pl.BlockSpec((br, D), lambda i: (i, 0)),
        out_shape=jax.ShapeDtypeStruct((nrows, D), jnp.float32),
    )(accp, selfh, nin, nout, w, b2d)


def kernel(x, edge_index, W1, b1, W2, b2):
    src = edge_index[0]
    dst = edge_index[1]
    pad = EP - E
    padidx = N + (jnp.arange(pad, dtype=jnp.int32) % (ROWS - N))
    src2d = jnp.concatenate([src, padidx]).reshape(EP // B, B)
    dst2d = jnp.concatenate([dst, padidx]).reshape(EP // B, B)
    xp = jnp.concatenate([x, jnp.zeros((ROWS - N, D), jnp.float32)])
    zrows = jnp.zeros((B, D), jnp.float32)

    degp = _deg_kernel(src2d, dst2d)
    degt = degp.reshape(4, ROWS).T
    h1s, nin, nout = _tc_prep(degt, xp)

    acc1 = _agg_kernel(h1s, src2d, dst2d, zrows)
    h2s = _tc_combine(acc1, h1s, nin, nout, W1, b1.reshape(1, D), last=False)

    acc2 = _agg_kernel(h2s, src2d, dst2d, zrows)
    out = _tc_combine(acc2, h2s, nin, nout, W2, b2.reshape(1, D), last=True)
    return out


# speedup vs baseline: 1.0145x; 1.0145x over previous
import functools

import jax
import jax.numpy as jnp
from jax import lax
from jax.experimental import pallas as pl
from jax.experimental.pallas import tpu as pltpu
from jax.experimental.pallas import tpu_sc as plsc

N = 10000
D = 128
E = 320000

NC = 2
NS = 16
NW = NC * NS
L = 16

B = 128
G = 80
EPW = G * B
EP = NW * EPW
GH = G // 2
ROWS = 10112
RPT = ROWS // NS

_sc_mesh = plsc.VectorSubcoreMesh(core_axis_name="c", subcore_axis_name="s")


@functools.partial(
    pl.kernel,
    out_type=jax.ShapeDtypeStruct((NC * 2 * ROWS,), jnp.float32),
    mesh=_sc_mesh,
    scratch_types=[
        pltpu.VMEM((G, B), jnp.int32),
        pltpu.VMEM((G, B), jnp.int32),
        pltpu.VMEM((B,), jnp.float32),
        pltpu.VMEM((640,), jnp.float32),
        pltpu.SemaphoreType.DMA,
        pltpu.SemaphoreType.DMA,
        pltpu.VMEM_SHARED((ROWS,), jnp.float32),
        pltpu.VMEM_SHARED((ROWS,), jnp.float32),
    ],
)
def _deg_kernel(src_hbm, dst_hbm, out_hbm, sidx, didx, ones, zbuf,
                sema, semb, acc_o, acc_i):
    c = lax.axis_index("c")
    s = lax.axis_index("s")
    wid = s * NC + c

    vone = jnp.ones((L,), jnp.float32)
    vz = jnp.zeros((L,), jnp.float32)
    for j in range(B // L):
        ones[pl.ds(j * L, L)] = vone
    for j in range(640 // L):
        zbuf[pl.ds(j * L, L)] = vz

    r0 = s * RPT
    pltpu.sync_copy(zbuf.at[pl.ds(0, RPT)], acc_o.at[pl.ds(r0, RPT)])
    pltpu.sync_copy(zbuf.at[pl.ds(0, RPT)], acc_i.at[pl.ds(r0, RPT)])
    pltpu.sync_copy(src_hbm.at[pl.ds(wid * G, G)], sidx)
    pltpu.sync_copy(dst_hbm.at[pl.ds(wid * G, G)], didx)
    plsc.subcore_barrier()

    pltpu.async_copy(ones, acc_o.at[sidx.at[0]], sema, add=True)
    pltpu.async_copy(ones, acc_i.at[didx.at[0]], semb, add=True)

    def body(g, _):
        pltpu.async_copy(ones, acc_o.at[sidx.at[g]], sema, add=True)
        pltpu.async_copy(ones, acc_i.at[didx.at[g]], semb, add=True)
        pltpu.make_async_copy(ones, acc_o.at[sidx.at[g]], sema).wait()
        pltpu.make_async_copy(ones, acc_i.at[didx.at[g]], semb).wait()
        return 0

    lax.fori_loop(1, G, body, 0)
    pltpu.make_async_copy(ones, acc_o.at[sidx.at[0]], sema).wait()
    pltpu.make_async_copy(ones, acc_i.at[didx.at[0]], semb).wait()
    plsc.subcore_barrier()

    pltpu.sync_copy(acc_o.at[pl.ds(r0, RPT)], zbuf.at[pl.ds(0, RPT)])
    pltpu.sync_copy(zbuf.at[pl.ds(0, RPT)],
                    out_hbm.at[pl.ds((c * 2) * ROWS + r0, RPT)])
    pltpu.sync_copy(acc_i.at[pl.ds(r0, RPT)], zbuf.at[pl.ds(0, RPT)])
    pltpu.sync_copy(zbuf.at[pl.ds(0, RPT)],
                    out_hbm.at[pl.ds((c * 2 + 1) * ROWS + r0, RPT)])


@functools.partial(
    pl.kernel,
    out_type=jax.ShapeDtypeStruct((NC, ROWS, D), jnp.float32),
    mesh=_sc_mesh,
    scratch_types=[
        pltpu.VMEM((GH, B), jnp.int32),
        pltpu.VMEM((GH, B), jnp.int32),
        pltpu.VMEM((B, D), jnp.float32),
        pltpu.VMEM((B, D), jnp.float32),
        pltpu.SemaphoreType.DMA,
        pltpu.SemaphoreType.DMA,
        pltpu.SemaphoreType.DMA,
        pltpu.SemaphoreType.DMA,
        pltpu.VMEM_SHARED((ROWS, D), jnp.float32),
    ],
)
def _agg_kernel(h_hbm, src_hbm, dst_hbm, z_hbm, out_hbm,
                sidx, didx, rows0, rows1, sem0, sem1, ssem0, ssem1, acc):
    c = lax.axis_index("c")
    s = lax.axis_index("s")
    wid = s * NC + c

    r0 = s * RPT
    pltpu.sync_copy(z_hbm, rows0)
    for p in range(RPT // B):
        pltpu.sync_copy(rows0, acc.at[pl.ds(r0 + p * B, B)])
    rem = RPT % B
    if rem:
        pltpu.sync_copy(rows0.at[pl.ds(0, rem)],
                        acc.at[pl.ds(r0 + (RPT // B) * B, rem)])
    plsc.subcore_barrier()

    for half in range(2):
        pltpu.sync_copy(src_hbm.at[pl.ds(wid * G + half * GH, GH)], sidx)
        pltpu.sync_copy(dst_hbm.at[pl.ds(wid * G + half * GH, GH)], didx)

        pltpu.async_copy(h_hbm.at[sidx.at[0]], rows0, sem0)

        def body(k, _):
            g = 2 * k
            pltpu.async_copy(h_hbm.at[sidx.at[g + 1]], rows1, sem1)
            pltpu.make_async_copy(h_hbm.at[sidx.at[g]], rows0, sem0).wait()
            pltpu.sync_copy(rows0, acc.at[didx.at[g]], add=True)
            pltpu.async_copy(h_hbm.at[sidx.at[g + 2]], rows0, sem0)
            pltpu.make_async_copy(h_hbm.at[sidx.at[g + 1]], rows1, sem1).wait()
            pltpu.sync_copy(rows1, acc.at[didx.at[g + 1]], add=True)
            return 0

        lax.fori_loop(0, GH // 2 - 1, body, 0)
        pltpu.async_copy(h_hbm.at[sidx.at[GH - 1]], rows1, sem1)
        pltpu.make_async_copy(h_hbm.at[sidx.at[GH - 2]], rows0, sem0).wait()
        pltpu.sync_copy(rows0, acc.at[didx.at[GH - 2]], add=True)
        pltpu.make_async_copy(h_hbm.at[sidx.at[GH - 1]], rows1, sem1).wait()
        pltpu.sync_copy(rows1, acc.at[didx.at[GH - 1]], add=True)

    plsc.subcore_barrier()
    for p in range(RPT // B):
        pltpu.sync_copy(acc.at[pl.ds(r0 + p * B, B)], rows0)
        pltpu.sync_copy(rows0, out_hbm.at[c, pl.ds(r0 + p * B, B)])
    if RPT % B:
        q = (RPT // B) * B
        pltpu.sync_copy(acc.at[pl.ds(r0 + q, RPT % B)],
                        rows0.at[pl.ds(0, RPT % B)])
        pltpu.sync_copy(rows0.at[pl.ds(0, RPT % B)],
                        out_hbm.at[c, pl.ds(r0 + q, RPT % B)])


BR = 1264
GRID = ROWS // BR


def _prep_body(degp_ref, x_ref, h1s_ref, nin_ref, nout_ref):
    dt = jnp.transpose(degp_ref[...], (1, 0))
    deg_out = dt[:, 0:1] + dt[:, 2:3] + 1.0
    deg_in = dt[:, 1:2] + dt[:, 3:4] + 1.0
    nout = lax.rsqrt(deg_out)
    nin = lax.rsqrt(deg_in)
    h1s_ref[...] = x_ref[...] * nout
    nin_ref[...] = jnp.broadcast_to(nin, (ROWS, D))
    nout_ref[...] = jnp.broadcast_to(nout, (ROWS, D))


def _tc_prep(degp2, xp):
    return pl.pallas_call(
        _prep_body,
        out_shape=[
            jax.ShapeDtypeStruct((ROWS, D), jnp.float32),
            jax.ShapeDtypeStruct((ROWS, D), jnp.float32),
            jax.ShapeDtypeStruct((ROWS, D), jnp.float32),
        ],
    )(degp2, xp)


def _combine_body(accp_ref, selfh_ref, nin_ref, nout_ref, w_ref, b_ref,
                  out_ref, *, last, br):
    agg = accp_ref[0] + accp_ref[1] + selfh_ref[...]
    t = agg * nin_ref[...]
    y = jnp.dot(t, w_ref[...], preferred_element_type=jnp.float32) + b_ref[...]
    if last:
        out_ref[...] = y
    else:
        out_ref[...] = jnp.maximum(y, 0.0) * nout_ref[...]


def _tc_combine(accp, selfh, nin, nout, w, b2d, last):
    nrows, br = (N, 1000) if last else (ROWS, BR)
    return pl.pallas_call(
        functools.partial(_combine_body, last=last, br=br),
        grid=(nrows // br,),
        in_specs=[
            pl.BlockSpec((NC, br, D), lambda i: (0, i, 0)),
            pl.BlockSpec((br, D), lambda i: (i, 0)),
            pl.BlockSpec((br, D), lambda i: (i, 0)),
            pl.BlockSpec((br, D), lambda i: (i, 0)),
            pl.BlockSpec((D, D), lambda i: (0, 0)),
            pl.BlockSpec((1, D), lambda i: (0, 0)),
        ],
        out_specs=pl.BlockSpec((br, D), lambda i: (i, 0)),
        out_shape=jax.ShapeDtypeStruct((nrows, D), jnp.float32),
    )(accp, selfh, nin, nout, w, b2d)


def kernel(x, edge_index, W1, b1, W2, b2):
    src = edge_index[0]
    dst = edge_index[1]
    pad = EP - E
    padidx = N + (jnp.arange(pad, dtype=jnp.int32) % (ROWS - N))
    src2d = jnp.concatenate([src, padidx]).reshape(EP // B, B)
    dst2d = jnp.concatenate([dst, padidx]).reshape(EP // B, B)
    xp = jnp.concatenate([x, jnp.zeros((ROWS - N, D), jnp.float32)])
    zrows = jnp.zeros((B, D), jnp.float32)

    degp = _deg_kernel(src2d, dst2d)
    degp2 = degp.reshape(4, ROWS)
    h1s, nin, nout = _tc_prep(degp2, xp)

    acc1 = _agg_kernel(h1s, src2d, dst2d, zrows)
    h2s = _tc_combine(acc1, h1s, nin, nout, W1, b1.reshape(1, D), last=False)

    acc2 = _agg_kernel(h2s, src2d, dst2d, zrows)
    out = _tc_combine(acc2, h2s, nin, nout, W2, b2.reshape(1, D), last=True)
    return out
